# baseline (speedup 1.0000x reference)
import functools

import jax
import jax.numpy as jnp
from jax import lax
from jax.experimental import pallas as pl
from jax.experimental.pallas import tpu as pltpu
from jax.experimental.pallas import tpu_sc as plsc

S, B, D = 2048, 4, 2048
NC, NS = 2, 16
NW = NC * NS
ROWS_W = S // NW
CHUNK = 4
NCHUNK = ROWS_W // CHUNK
NBUF = 3


@functools.partial(
    pl.kernel,
    mesh=plsc.VectorSubcoreMesh(core_axis_name="c", subcore_axis_name="s"),
    out_type=jax.ShapeDtypeStruct((S + 1, B, D), jnp.float32),
    scratch_types=[
        pltpu.VMEM((NBUF, CHUNK, B, D), jnp.float32),
        pltpu.VMEM((B,), jnp.int32),
        pltpu.VMEM((B, D), jnp.float32),
        pltpu.SemaphoreType.DMA((NBUF,)),
        pltpu.SemaphoreType.DMA((NBUF,)),
        pltpu.SemaphoreType.DMA,
    ],
)
def _sc_concat_embed(x_hbm, t_hbm, table_hbm, out_hbm,
                     buf, idx_v, rows_v, in_sems, out_sems, gsem):
    wid = lax.axis_index("s") * NC + lax.axis_index("c")
    base = wid * ROWS_W

    @pl.when(wid == 0)
    def _embed():
        pltpu.sync_copy(t_hbm, idx_v)
        pltpu.async_copy(table_hbm.at[idx_v], rows_v, gsem).wait()
        pltpu.sync_copy(rows_v, out_hbm.at[S])

    def in_cp(k, slot):
        return pltpu.make_async_copy(
            x_hbm.at[pl.ds(base + k * CHUNK, CHUNK)], buf.at[slot],
            in_sems.at[slot])

    def out_cp(k, slot):
        return pltpu.make_async_copy(
            buf.at[slot], out_hbm.at[pl.ds(base + k * CHUNK, CHUNK)],
            out_sems.at[slot])

    for k in range(min(NBUF, NCHUNK)):
        in_cp(k, k).start()
    for k in range(NCHUNK):
        slot = k % NBUF
        in_cp(k, slot).wait()
        out_cp(k, slot).start()
        nk = k + NBUF
        if nk < NCHUNK:
            out_cp(k, slot).wait()
            in_cp(nk, slot).start()
    for k in range(max(0, NCHUNK - NBUF), NCHUNK):
        out_cp(k, k % NBUF).wait()


def kernel(x, t, table):
    return _sc_concat_embed(x, t, table)


# speedup vs baseline: 1.0459x; 1.0459x over previous
import functools

import jax
import jax.numpy as jnp
from jax import lax
from jax.experimental import pallas as pl
from jax.experimental.pallas import tpu as pltpu
from jax.experimental.pallas import tpu_sc as plsc

S, B, D = 2048, 4, 2048
BS = 128
N = S // BS


@functools.partial(
    pl.kernel,
    mesh=plsc.VectorSubcoreMesh(core_axis_name="c", subcore_axis_name="s"),
    out_type=jax.ShapeDtypeStruct((B, D), jnp.float32),
    scratch_types=[
        pltpu.VMEM((B,), jnp.int32),
        pltpu.VMEM((B, D), jnp.float32),
        pltpu.SemaphoreType.DMA,
    ],
)
def _sc_embed(t_hbm, table_hbm, emb_hbm, idx_v, rows_v, gsem):
    wid = lax.axis_index("s") * 2 + lax.axis_index("c")

    @pl.when(wid == 0)
    def _gather():
        pltpu.sync_copy(t_hbm, idx_v)
        pltpu.async_copy(table_hbm.at[idx_v], rows_v, gsem).wait()
        pltpu.sync_copy(rows_v, emb_hbm)


def _tc_concat_body(x_ref, emb_ref, out_ref):
    i = pl.program_id(0)

    @pl.when(i < N)
    def _copy():
        out_ref[...] = x_ref[...]

    @pl.when(i == N)
    def _last():
        out_ref[0] = emb_ref[...]


def kernel(x, t, table):
    t_emb = _sc_embed(t, table)
    return pl.pallas_call(
        _tc_concat_body,
        grid=(N + 1,),
        out_shape=jax.ShapeDtypeStruct((S + 1, B, D), x.dtype),
        in_specs=[
            pl.BlockSpec((BS, B, D), lambda i: (jnp.minimum(i, N - 1), 0, 0)),
            pl.BlockSpec((B, D), lambda i: (0, 0)),
        ],
        out_specs=pl.BlockSpec((BS, B, D), lambda i: (i, 0, 0)),
    )(x, t_emb)
